# Initial kernel scaffold; baseline (speedup 1.0000x reference)
import jax
import jax.numpy as jnp
from jax.experimental import pallas as pl


def kernel(input, head_emb, head_W, tail_emb0, tail_W0, tail_emb1, tail_W1):
    raise NotImplementedError("write your pallas kernel here")


# speedup vs baseline: 2.9767x; 2.9767x over previous
import functools

import jax
import jax.numpy as jnp
from jax import lax
from jax.experimental import pallas as pl
from jax.experimental.pallas import tpu as pltpu
from jax.experimental.pallas import tpu_sc as plsc

EMB = 64
C0, C1, C2 = 20000, 200000, 1000000
N = 4096 * 200

NC, NS = 2, 16
NW = NC * NS
TOK_W = N // NW
CH = 512
NCH = TOK_W // CH
GB = 128

_mesh = plsc.VectorSubcoreMesh(
    core_axis_name="c", subcore_axis_name="s", num_cores=NC, num_subcores=NS
)


@functools.partial(
    pl.kernel,
    mesh=_mesh,
    compiler_params=pltpu.CompilerParams(use_tc_tiling_on_sc=False),
    out_type=(
        jax.ShapeDtypeStruct((N, 64), jnp.float32),
        jax.ShapeDtypeStruct((N, 16), jnp.float32),
        jax.ShapeDtypeStruct((N, 16), jnp.float32),
    ),
    scratch_types=[
        pltpu.VMEM((CH,), jnp.int32),
        pltpu.VMEM((CH,), jnp.int32),
        pltpu.VMEM((CH,), jnp.int32),
        pltpu.VMEM((CH,), jnp.int32),
        pltpu.VMEM((CH, 64), jnp.float32),
        pltpu.VMEM((CH, 16), jnp.float32),
        pltpu.VMEM((CH, 16), jnp.float32),
        pltpu.SemaphoreType.DMA,
    ],
)
def _sc_gather(flat_hbm, head_hbm, t0_hbm, t1_hbm, r0_hbm, r1_hbm, r2_hbm,
               idx_v, i0_v, i1_v, i2_v, rows0_v, rows1_v, rows2_v, sem):
    wid = lax.axis_index("s") * NC + lax.axis_index("c")
    base_w = wid * TOK_W

    def chunk_body(g, carry):
        base = base_w + g * CH
        pltpu.sync_copy(flat_hbm.at[pl.ds(base, CH)], idx_v)

        def vec_body(j, c):
            v = idx_v[pl.ds(j * 16, 16)]
            i0_v[pl.ds(j * 16, 16)] = jnp.minimum(v, C0 - 1)
            i1_v[pl.ds(j * 16, 16)] = jnp.clip(v - C0, 0, C1 - C0 - 1)
            i2_v[pl.ds(j * 16, 16)] = jnp.clip(v - C1, 0, C2 - C1 - 1)
            return c

        lax.fori_loop(0, CH // 16, vec_body, 0)

        copies = []
        for b in range(CH // GB):
            s = pl.ds(b * GB, GB)
            copies.append(pltpu.async_copy(
                head_hbm.at[i0_v.at[s]], rows0_v.at[s], sem))
            copies.append(pltpu.async_copy(
                t0_hbm.at[i1_v.at[s]], rows1_v.at[s], sem))
            copies.append(pltpu.async_copy(
                t1_hbm.at[i2_v.at[s]], rows2_v.at[s], sem))
        for cp in copies:
            cp.wait()

        dst = pl.ds(base, CH)
        pltpu.sync_copy(rows0_v, r0_hbm.at[dst])
        pltpu.sync_copy(rows1_v, r1_hbm.at[dst])
        pltpu.sync_copy(rows2_v, r2_hbm.at[dst])
        return carry

    lax.fori_loop(0, NCH, chunk_body, 0)


TB = 1024


def _tc_body(idx_ref, r0_ref, r1_ref, r2_ref, hw_ref, w0_ref, w1_ref, out_ref):
    idx = idx_ref[...]
    m0 = (idx < C0).astype(jnp.float32)
    m1 = ((idx >= C0) & (idx < C1)).astype(jnp.float32)
    m2 = (idx >= C1).astype(jnp.float32)
    a0 = r0_ref[...] * m0
    a1 = r1_ref[...] * m1
    a2 = r2_ref[...] * m2
    out_ref[...] = (
        jnp.dot(a0, hw_ref[...], preferred_element_type=jnp.float32)
        + jnp.dot(a1, w0_ref[...], preferred_element_type=jnp.float32)
        + jnp.dot(a2, w1_ref[...], preferred_element_type=jnp.float32)
    )


def kernel(input, head_emb, head_W, tail_emb0, tail_W0, tail_emb1, tail_W1):
    flat = input.reshape(-1).astype(jnp.int32)
    t1p = jnp.pad(tail_emb1, ((0, 0), (0, 12)))
    w1p = jnp.pad(tail_W1, ((0, 12), (0, 0)))
    r0, r1, r2 = _sc_gather(flat, head_emb, tail_emb0, t1p)

    out = pl.pallas_call(
        _tc_body,
        grid=(N // TB,),
        in_specs=[
            pl.BlockSpec((TB, 1), lambda i: (i, 0)),
            pl.BlockSpec((TB, 64), lambda i: (i, 0)),
            pl.BlockSpec((TB, 16), lambda i: (i, 0)),
            pl.BlockSpec((TB, 16), lambda i: (i, 0)),
            pl.BlockSpec((64, 64), lambda i: (0, 0)),
            pl.BlockSpec((16, 64), lambda i: (0, 0)),
            pl.BlockSpec((16, 64), lambda i: (0, 0)),
        ],
        out_specs=pl.BlockSpec((TB, 64), lambda i: (i, 0)),
        out_shape=jax.ShapeDtypeStruct((N, 64), jnp.float32),
    )(flat.reshape(N, 1), r0, r1, r2, head_W, tail_W0, w1p)

    return out.reshape(input.shape[0], input.shape[1], EMB)


# speedup vs baseline: 2.9830x; 1.0021x over previous
import functools

import jax
import jax.numpy as jnp
from jax import lax
from jax.experimental import pallas as pl
from jax.experimental.pallas import tpu as pltpu
from jax.experimental.pallas import tpu_sc as plsc

EMB = 64
C0, C1, C2 = 20000, 200000, 1000000
N = 4096 * 200

NC, NS = 2, 16
NW = NC * NS
TOK_W = N // NW
CH = 512
NCH = TOK_W // CH
GB = 512

_mesh = plsc.VectorSubcoreMesh(
    core_axis_name="c", subcore_axis_name="s", num_cores=NC, num_subcores=NS
)


@functools.partial(
    pl.kernel,
    mesh=_mesh,
    compiler_params=pltpu.CompilerParams(use_tc_tiling_on_sc=False),
    out_type=(
        jax.ShapeDtypeStruct((N, 64), jnp.float32),
        jax.ShapeDtypeStruct((N, 16), jnp.float32),
        jax.ShapeDtypeStruct((N, 16), jnp.float32),
    ),
    scratch_types=[
        pltpu.VMEM((CH,), jnp.int32),
        pltpu.VMEM((CH,), jnp.int32),
        pltpu.VMEM((CH,), jnp.int32),
        pltpu.VMEM((CH,), jnp.int32),
        pltpu.VMEM((CH, 64), jnp.float32),
        pltpu.VMEM((CH, 16), jnp.float32),
        pltpu.VMEM((CH, 16), jnp.float32),
        pltpu.SemaphoreType.DMA,
    ],
)
def _sc_gather(flat_hbm, head_hbm, t0_hbm, t1_hbm, r0_hbm, r1_hbm, r2_hbm,
               idx_v, i0_v, i1_v, i2_v, rows0_v, rows1_v, rows2_v, sem):
    wid = lax.axis_index("s") * NC + lax.axis_index("c")
    base_w = wid * TOK_W

    def chunk_body(g, carry):
        base = base_w + g * CH
        pltpu.sync_copy(flat_hbm.at[pl.ds(base, CH)], idx_v)

        def vec_body(j, c):
            v = idx_v[pl.ds(j * 16, 16)]
            i0_v[pl.ds(j * 16, 16)] = jnp.minimum(v, C0 - 1)
            i1_v[pl.ds(j * 16, 16)] = jnp.clip(v - C0, 0, C1 - C0 - 1)
            i2_v[pl.ds(j * 16, 16)] = jnp.clip(v - C1, 0, C2 - C1 - 1)
            return c

        lax.fori_loop(0, CH // 16, vec_body, 0)

        copies = []
        for b in range(CH // GB):
            s = pl.ds(b * GB, GB)
            copies.append(pltpu.async_copy(
                head_hbm.at[i0_v.at[s]], rows0_v.at[s], sem))
            copies.append(pltpu.async_copy(
                t0_hbm.at[i1_v.at[s]], rows1_v.at[s], sem))
            copies.append(pltpu.async_copy(
                t1_hbm.at[i2_v.at[s]], rows2_v.at[s], sem))
        for cp in copies:
            cp.wait()

        dst = pl.ds(base, CH)
        pltpu.sync_copy(rows0_v, r0_hbm.at[dst])
        pltpu.sync_copy(rows1_v, r1_hbm.at[dst])
        pltpu.sync_copy(rows2_v, r2_hbm.at[dst])
        return carry

    lax.fori_loop(0, NCH, chunk_body, 0)


TB = 1024


def _tc_body(idx_ref, r0_ref, r1_ref, r2_ref, hw_ref, w0_ref, w1_ref, out_ref):
    idx = idx_ref[...]
    m0 = (idx < C0).astype(jnp.float32)
    m1 = ((idx >= C0) & (idx < C1)).astype(jnp.float32)
    m2 = (idx >= C1).astype(jnp.float32)
    a0 = r0_ref[...] * m0
    a1 = r1_ref[...] * m1
    a2 = r2_ref[...] * m2
    out_ref[...] = (
        jnp.dot(a0, hw_ref[...], preferred_element_type=jnp.float32)
        + jnp.dot(a1, w0_ref[...], preferred_element_type=jnp.float32)
        + jnp.dot(a2, w1_ref[...], preferred_element_type=jnp.float32)
    )


def kernel(input, head_emb, head_W, tail_emb0, tail_W0, tail_emb1, tail_W1):
    flat = input.reshape(-1).astype(jnp.int32)
    t1p = jnp.pad(tail_emb1, ((0, 0), (0, 12)))
    w1p = jnp.pad(tail_W1, ((0, 12), (0, 0)))
    r0, r1, r2 = _sc_gather(flat, head_emb, tail_emb0, t1p)

    out = pl.pallas_call(
        _tc_body,
        grid=(N // TB,),
        in_specs=[
            pl.BlockSpec((TB, 1), lambda i: (i, 0)),
            pl.BlockSpec((TB, 64), lambda i: (i, 0)),
            pl.BlockSpec((TB, 16), lambda i: (i, 0)),
            pl.BlockSpec((TB, 16), lambda i: (i, 0)),
            pl.BlockSpec((64, 64), lambda i: (0, 0)),
            pl.BlockSpec((16, 64), lambda i: (0, 0)),
            pl.BlockSpec((16, 64), lambda i: (0, 0)),
        ],
        out_specs=pl.BlockSpec((TB, 64), lambda i: (i, 0)),
        out_shape=jax.ShapeDtypeStruct((N, 64), jnp.float32),
    )(flat.reshape(N, 1), r0, r1, r2, head_W, tail_W0, w1p)

    return out.reshape(input.shape[0], input.shape[1], EMB)


# speedup vs baseline: 14.9672x; 5.0175x over previous
import functools

import jax
import jax.numpy as jnp
from jax import lax
from jax.experimental import pallas as pl
from jax.experimental.pallas import tpu as pltpu
from jax.experimental.pallas import tpu_sc as plsc

EMB = 64
C0, C1, C2 = 20000, 200000, 1000000
T0ROWS = C1 - C0
TGROWS = (C2 - C1) // 4
N = 4096 * 200

NC, NS = 2, 16
NW = NC * NS
TOK_W = N // NW
CH = 512
NCH = TOK_W // CH
HB = 128

_mesh = plsc.VectorSubcoreMesh(
    core_axis_name="c", subcore_axis_name="s", num_cores=NC, num_subcores=NS
)
_sc_params = pltpu.CompilerParams(use_tc_tiling_on_sc=False, needs_layout_passes=False)


@functools.partial(
    pl.kernel,
    mesh=_mesh,
    compiler_params=_sc_params,
    out_type=jax.ShapeDtypeStruct((N, 16), jnp.float32),
    scratch_types=[
        pltpu.VMEM((CH,), jnp.int32),
        pltpu.VMEM((CH,), jnp.int32),
        pltpu.VMEM((CH, 16), jnp.float32),
        pltpu.SemaphoreType.DMA,
    ],
)
def _sc_gather(flat_hbm, comb_hbm, r12_hbm, idx_v, lidx_v, rows_v, sem):
    wid = lax.axis_index("s") * NC + lax.axis_index("c")
    base_w = wid * TOK_W

    def chunk_body(g, carry):
        base = base_w + g * CH
        pltpu.sync_copy(flat_hbm.at[pl.ds(base, CH)], idx_v)

        def vec_body(j, c):
            v = idx_v[pl.ds(j * 16, 16)]
            m1 = v < C1
            l1 = v - C0
            l2 = T0ROWS + ((v - C1) >> 2)
            lidx = jnp.where(v < C0, 0, jnp.where(m1, l1, l2))
            lidx_v[pl.ds(j * 16, 16)] = lidx
            return c

        lax.fori_loop(0, CH // 16, vec_body, 0)
        pltpu.async_copy(comb_hbm.at[lidx_v], rows_v, sem).wait()
        pltpu.sync_copy(rows_v, r12_hbm.at[pl.ds(base, CH)])
        return carry

    lax.fori_loop(0, NCH, chunk_body, 0)


LCAP = TOK_W + HB + 16
DUMP = LCAP - 1


@functools.partial(
    pl.kernel,
    mesh=_mesh,
    compiler_params=_sc_params,
    out_type=(),
    scratch_types=[
        pltpu.VMEM((CH,), jnp.int32),
        pltpu.VMEM((LCAP,), jnp.int32),
        pltpu.VMEM((LCAP,), jnp.int32),
        pltpu.VMEM((HB, EMB), jnp.float32),
        pltpu.SemaphoreType.DMA,
    ],
)
def _sc_head_scatter(flat_hbm, p0_hbm, out_ref, idx_v, pos_v, hidx_v, rows_v, sem):
    wid = lax.axis_index("s") * NC + lax.axis_index("c")
    base_w = wid * TOK_W

    def chunk_body(g, off):
        base = base_w + g * CH
        pltpu.sync_copy(flat_hbm.at[pl.ds(base, CH)], idx_v)

        def vec_body(j, off):
            v = idx_v[pl.ds(j * 16, 16)]
            m0 = v < C0
            c = jnp.where(m0, 1, 0)
            rank = plsc.cumsum(c) - c
            dst = jnp.where(m0, off + rank, DUMP)
            plsc.store_scatter(pos_v, [dst], base + j * 16 + lax.iota(jnp.int32, 16))
            plsc.store_scatter(hidx_v, [dst], v)
            return off + jnp.sum(c)

        return lax.fori_loop(0, CH // 16, vec_body, off)

    off = lax.fori_loop(0, NCH, chunk_body, 0)

    nb = (off + HB - 1) // HB
    offp = nb * HB

    @pl.when(off > 0)
    def _pad():
        lane0 = lax.iota(jnp.int32, 16) == 0
        pos0 = jnp.sum(jnp.where(lane0, pos_v[pl.ds(0, 16)], 0))
        hid0 = jnp.sum(jnp.where(lane0, hidx_v[pl.ds(0, 16)], 0))

        def pad_body(k, c):
            lanes = k * 16 + lax.iota(jnp.int32, 16)
            m = lanes >= off
            cur_p = pos_v[pl.ds(k * 16, 16)]
            cur_h = hidx_v[pl.ds(k * 16, 16)]
            pos_v[pl.ds(k * 16, 16)] = jnp.where(m, pos0, cur_p)
            hidx_v[pl.ds(k * 16, 16)] = jnp.where(m, hid0, cur_h)
            return c

        lax.fori_loop(off // 16, (offp + 15) // 16, pad_body, 0)

    def batch_body(b, c):
        s = pl.ds(b * HB, HB)
        pltpu.async_copy(p0_hbm.at[hidx_v.at[s]], rows_v, sem).wait()
        pltpu.async_copy(rows_v, out_ref.at[pos_v.at[s]], sem).wait()
        return c

    lax.fori_loop(0, nb, batch_body, 0)


def _tc_p0_body(emb_ref, w_ref, out_ref):
    out_ref[...] = jnp.dot(emb_ref[...], w_ref[...],
                           preferred_element_type=jnp.float32)


TB = 1024


def _tc_body(idx_ref, r12_ref, w0_ref, w1_ref, out_ref):
    idx = idx_ref[...]
    r12 = r12_ref[...]
    m1 = (idx >= C0) & (idx < C1)
    a1 = jnp.where(m1, r12, 0.0)
    m2 = idx >= C1
    rem = (idx - C1) & 3
    a2 = jnp.zeros((TB, 4), jnp.float32)
    for l in range(4):
        a2 = a2 + jnp.where(m2 & (rem == l), r12[:, 4 * l:4 * l + 4], 0.0)
    out_ref[...] = (
        jnp.dot(a1, w0_ref[...], preferred_element_type=jnp.float32)
        + jnp.dot(a2, w1_ref[...], preferred_element_type=jnp.float32)
    )


def kernel(input, head_emb, head_W, tail_emb0, tail_W0, tail_emb1, tail_W1):
    flat = input.reshape(-1).astype(jnp.int32)
    comb = jnp.concatenate(
        [tail_emb0, tail_emb1.reshape(TGROWS, 16)], axis=0)

    p0 = pl.pallas_call(
        _tc_p0_body,
        grid=(10,),
        in_specs=[
            pl.BlockSpec((C0 // 10, EMB), lambda i: (i, 0)),
            pl.BlockSpec((EMB, EMB), lambda i: (0, 0)),
        ],
        out_specs=pl.BlockSpec((C0 // 10, EMB), lambda i: (i, 0)),
        out_shape=jax.ShapeDtypeStruct((C0, EMB), jnp.float32),
    )(head_emb, head_W)

    r12 = _sc_gather(flat, comb)

    out = pl.pallas_call(
        _tc_body,
        grid=(N // TB,),
        in_specs=[
            pl.BlockSpec((TB, 1), lambda i: (i, 0)),
            pl.BlockSpec((TB, 16), lambda i: (i, 0)),
            pl.BlockSpec((16, EMB), lambda i: (0, 0)),
            pl.BlockSpec((4, EMB), lambda i: (0, 0)),
        ],
        out_specs=pl.BlockSpec((TB, EMB), lambda i: (i, 0)),
        out_shape=jax.ShapeDtypeStruct((N, EMB), jnp.float32),
    )(flat.reshape(N, 1), r12, tail_W0, tail_W1)

    ref = jax.new_ref(out)
    _sc_head_scatter(flat, p0, ref)
    return ref[...].reshape(input.shape[0], input.shape[1], EMB)


# speedup vs baseline: 17.9790x; 1.2012x over previous
import functools

import jax
import jax.numpy as jnp
from jax import lax
from jax.experimental import pallas as pl
from jax.experimental.pallas import tpu as pltpu
from jax.experimental.pallas import tpu_sc as plsc

EMB = 64
C0, C1, C2 = 20000, 200000, 1000000
T0ROWS = C1 - C0
TGROWS = (C2 - C1) // 4
TROWS = T0ROWS + TGROWS
N = 4096 * 200

NC, NS = 2, 16
NW = NC * NS
TOK_W = N // NW
CH = 512
NCH = TOK_W // CH
HB = 128

_mesh = plsc.VectorSubcoreMesh(
    core_axis_name="c", subcore_axis_name="s", num_cores=NC, num_subcores=NS
)
_sc_params = pltpu.CompilerParams(use_tc_tiling_on_sc=False, needs_layout_passes=False)


@functools.partial(
    pl.kernel,
    mesh=_mesh,
    compiler_params=_sc_params,
    out_type=jax.ShapeDtypeStruct((N, 16), jnp.float32),
    scratch_types=[
        pltpu.VMEM((CH,), jnp.int32),
        pltpu.VMEM((CH,), jnp.int32),
        pltpu.VMEM((CH, 16), jnp.float32),
        pltpu.SemaphoreType.DMA,
    ],
)
def _sc_gather(flat_hbm, comb_hbm, r12_hbm, idx_v, lidx_v, rows_v, sem):
    wid = lax.axis_index("s") * NC + lax.axis_index("c")
    base_w = wid * TOK_W

    def chunk_body(g, carry):
        base = base_w + g * CH
        pltpu.sync_copy(flat_hbm.at[pl.ds(base, CH)], idx_v)

        def vec_body(j, c):
            v = idx_v[pl.ds(j * 16, 16)]
            m1 = v < C1
            l1 = v - C0
            l2 = T0ROWS + ((v - C1) >> 2)
            lidx = jnp.where(v < C0, 0, jnp.where(m1, l1, l2))
            lidx_v[pl.ds(j * 16, 16)] = lidx
            return c

        lax.fori_loop(0, CH // 16, vec_body, 0)
        pltpu.async_copy(comb_hbm.at[lidx_v], rows_v, sem).wait()
        pltpu.sync_copy(rows_v, r12_hbm.at[pl.ds(base, CH)])
        return carry

    lax.fori_loop(0, NCH, chunk_body, 0)


LCAP = TOK_W + HB + 16
DUMP = LCAP - 1


@functools.partial(
    pl.kernel,
    mesh=_mesh,
    compiler_params=_sc_params,
    out_type=(),
    scratch_types=[
        pltpu.VMEM((CH,), jnp.int32),
        pltpu.VMEM((LCAP,), jnp.int32),
        pltpu.VMEM((LCAP,), jnp.int32),
        pltpu.VMEM((HB, EMB), jnp.float32),
        pltpu.SemaphoreType.DMA,
    ],
)
def _sc_head_scatter(flat_hbm, p0_hbm, out_ref, idx_v, pos_v, hidx_v, rows_v, sem):
    wid = lax.axis_index("s") * NC + lax.axis_index("c")
    base_w = wid * TOK_W

    def chunk_body(g, off):
        base = base_w + g * CH
        pltpu.sync_copy(flat_hbm.at[pl.ds(base, CH)], idx_v)

        def vec_body(j, off):
            v = idx_v[pl.ds(j * 16, 16)]
            m0 = v < C0
            c = jnp.where(m0, 1, 0)
            rank = plsc.cumsum(c) - c
            dst = jnp.where(m0, off + rank, DUMP)
            plsc.store_scatter(pos_v, [dst], base + j * 16 + lax.iota(jnp.int32, 16))
            plsc.store_scatter(hidx_v, [dst], v)
            return off + jnp.sum(c)

        return lax.fori_loop(0, CH // 16, vec_body, off)

    off = lax.fori_loop(0, NCH, chunk_body, 0)

    nb = (off + HB - 1) // HB
    offp = nb * HB

    @pl.when(off > 0)
    def _pad():
        lane0 = lax.iota(jnp.int32, 16) == 0
        pos0 = jnp.sum(jnp.where(lane0, pos_v[pl.ds(0, 16)], 0))
        hid0 = jnp.sum(jnp.where(lane0, hidx_v[pl.ds(0, 16)], 0))

        def pad_body(k, c):
            lanes = k * 16 + lax.iota(jnp.int32, 16)
            m = lanes >= off
            cur_p = pos_v[pl.ds(k * 16, 16)]
            cur_h = hidx_v[pl.ds(k * 16, 16)]
            pos_v[pl.ds(k * 16, 16)] = jnp.where(m, pos0, cur_p)
            hidx_v[pl.ds(k * 16, 16)] = jnp.where(m, hid0, cur_h)
            return c

        lax.fori_loop(off // 16, (offp + 15) // 16, pad_body, 0)

    def batch_body(b, c):
        s = pl.ds(b * HB, HB)
        pltpu.async_copy(p0_hbm.at[hidx_v.at[s]], rows_v, sem).wait()
        pltpu.async_copy(rows_v, out_ref.at[pos_v.at[s]], sem).wait()
        return c

    lax.fori_loop(0, nb, batch_body, 0)


def _tc_p0_body(emb_ref, w_ref, out_ref):
    out_ref[...] = jnp.dot(emb_ref[...], w_ref[...],
                           preferred_element_type=jnp.float32)


TB = 1024


def _tc_body(idx_ref, r12_ref, w0_ref, w1_ref, out_ref):
    idx = idx_ref[...]
    r12 = r12_ref[...]
    m1 = (idx >= C0) & (idx < C1)
    a1 = jnp.where(m1, r12, 0.0)
    m2 = idx >= C1
    rem = (idx - C1) & 3
    a2 = jnp.zeros((TB, 4), jnp.float32)
    for l in range(4):
        a2 = a2 + jnp.where(m2 & (rem == l), r12[:, 4 * l:4 * l + 4], 0.0)
    out_ref[...] = (
        jnp.dot(a1, w0_ref[...], preferred_element_type=jnp.float32)
        + jnp.dot(a2, w1_ref[...], preferred_element_type=jnp.float32)
    )


def kernel(input, head_emb, head_W, tail_emb0, tail_W0, tail_emb1, tail_W1):
    flat = input.reshape(-1).astype(jnp.int32)
    t0part = jnp.transpose(
        tail_emb0.T.reshape(16, T0ROWS // 8, 8), (1, 2, 0)).reshape(T0ROWS // 8, 128)
    t1part = jnp.transpose(
        tail_emb1.T.reshape(4, TGROWS // 8, 8, 4), (1, 2, 3, 0)).reshape(TGROWS // 8, 128)
    comb128 = jnp.concatenate([t0part, t1part], axis=0)

    p0 = pl.pallas_call(
        _tc_p0_body,
        grid=(10,),
        in_specs=[
            pl.BlockSpec((C0 // 10, EMB), lambda i: (i, 0)),
            pl.BlockSpec((EMB, EMB), lambda i: (0, 0)),
        ],
        out_specs=pl.BlockSpec((C0 // 10, EMB), lambda i: (i, 0)),
        out_shape=jax.ShapeDtypeStruct((C0, EMB), jnp.float32),
    )(head_emb, head_W)

    r12 = _sc_gather(flat, comb128.reshape(TROWS, 16))

    out = pl.pallas_call(
        _tc_body,
        grid=(N // TB,),
        in_specs=[
            pl.BlockSpec((TB, 1), lambda i: (i, 0)),
            pl.BlockSpec((TB, 16), lambda i: (i, 0)),
            pl.BlockSpec((16, EMB), lambda i: (0, 0)),
            pl.BlockSpec((4, EMB), lambda i: (0, 0)),
        ],
        out_specs=pl.BlockSpec((TB, EMB), lambda i: (i, 0)),
        out_shape=jax.ShapeDtypeStruct((N, EMB), jnp.float32),
    )(flat.reshape(N, 1), r12, tail_W0, tail_W1)

    ref = jax.new_ref(out)
    _sc_head_scatter(flat, p0, ref)
    return ref[...].reshape(input.shape[0], input.shape[1], EMB)


# speedup vs baseline: 36.2150x; 2.0143x over previous
import functools

import jax
import jax.numpy as jnp
from jax import lax
from jax.experimental import pallas as pl
from jax.experimental.pallas import tpu as pltpu
from jax.experimental.pallas import tpu_sc as plsc

EMB = 64
C0, C1, C2 = 20000, 200000, 1000000
T0ROWS = C1 - C0
T1ROWS = C2 - C1
TROWS = T0ROWS + T1ROWS
N = 4096 * 200

NC, NS = 2, 16
NW = NC * NS
TOK_W = N // NW
CH = 512
NCH = TOK_W // CH
HB = 128

_mesh = plsc.VectorSubcoreMesh(
    core_axis_name="c", subcore_axis_name="s", num_cores=NC, num_subcores=NS
)
_sc_params = pltpu.CompilerParams(use_tc_tiling_on_sc=False, needs_layout_passes=False)


@functools.partial(
    pl.kernel,
    mesh=_mesh,
    compiler_params=_sc_params,
    out_type=jax.ShapeDtypeStruct((N, 16), jnp.float32),
    scratch_types=[
        pltpu.VMEM((CH,), jnp.int32),
        pltpu.VMEM((CH,), jnp.int32),
        pltpu.VMEM((CH, 16), jnp.float32),
        pltpu.SemaphoreType.DMA,
    ],
)
def _sc_gather(flat_hbm, comb_hbm, r12_hbm, idx_v, lidx_v, rows_v, sem):
    wid = lax.axis_index("s") * NC + lax.axis_index("c")
    base_w = wid * TOK_W

    def chunk_body(g, carry):
        base = base_w + g * CH
        pltpu.sync_copy(flat_hbm.at[pl.ds(base, CH)], idx_v)

        def vec_body(j, c):
            v = idx_v[pl.ds(j * 16, 16)]
            lidx = jnp.where(v < C0, 0,
                             jnp.where(v < C1, v - C0, T0ROWS + (v - C1)))
            lidx_v[pl.ds(j * 16, 16)] = lidx
            return c

        lax.fori_loop(0, CH // 16, vec_body, 0)
        pltpu.async_copy(comb_hbm.at[lidx_v], rows_v, sem).wait()
        pltpu.sync_copy(rows_v, r12_hbm.at[pl.ds(base, CH)])
        return carry

    lax.fori_loop(0, NCH, chunk_body, 0)


LCAP = TOK_W + HB + 16
DUMP = LCAP - 1


@functools.partial(
    pl.kernel,
    mesh=_mesh,
    compiler_params=_sc_params,
    out_type=(),
    scratch_types=[
        pltpu.VMEM((CH,), jnp.int32),
        pltpu.VMEM((LCAP,), jnp.int32),
        pltpu.VMEM((LCAP,), jnp.int32),
        pltpu.VMEM((HB, EMB), jnp.float32),
        pltpu.SemaphoreType.DMA,
    ],
)
def _sc_head_scatter(flat_hbm, p0_hbm, out_ref, idx_v, pos_v, hidx_v, rows_v, sem):
    wid = lax.axis_index("s") * NC + lax.axis_index("c")
    base_w = wid * TOK_W

    def chunk_body(g, off):
        base = base_w + g * CH
        pltpu.sync_copy(flat_hbm.at[pl.ds(base, CH)], idx_v)

        def vec_body(j, off):
            v = idx_v[pl.ds(j * 16, 16)]
            m0 = v < C0
            c = jnp.where(m0, 1, 0)
            rank = plsc.cumsum(c) - c
            dst = jnp.where(m0, off + rank, DUMP)
            plsc.store_scatter(pos_v, [dst], base + j * 16 + lax.iota(jnp.int32, 16))
            plsc.store_scatter(hidx_v, [dst], v)
            return off + jnp.sum(c)

        return lax.fori_loop(0, CH // 16, vec_body, off)

    off = lax.fori_loop(0, NCH, chunk_body, 0)

    nb = (off + HB - 1) // HB
    offp = nb * HB

    @pl.when(off > 0)
    def _pad():
        lane0 = lax.iota(jnp.int32, 16) == 0
        pos0 = jnp.sum(jnp.where(lane0, pos_v[pl.ds(0, 16)], 0))
        hid0 = jnp.sum(jnp.where(lane0, hidx_v[pl.ds(0, 16)], 0))

        def pad_body(k, c):
            lanes = k * 16 + lax.iota(jnp.int32, 16)
            m = lanes >= off
            cur_p = pos_v[pl.ds(k * 16, 16)]
            cur_h = hidx_v[pl.ds(k * 16, 16)]
            pos_v[pl.ds(k * 16, 16)] = jnp.where(m, pos0, cur_p)
            hidx_v[pl.ds(k * 16, 16)] = jnp.where(m, hid0, cur_h)
            return c

        lax.fori_loop(off // 16, (offp + 15) // 16, pad_body, 0)

    def batch_body(b, c):
        s = pl.ds(b * HB, HB)
        pltpu.async_copy(p0_hbm.at[hidx_v.at[s]], rows_v, sem).wait()
        pltpu.async_copy(rows_v, out_ref.at[pos_v.at[s]], sem).wait()
        return c

    lax.fori_loop(0, nb, batch_body, 0)


def _tc_p0_body(emb_ref, w_ref, out_ref):
    out_ref[...] = jnp.dot(emb_ref[...], w_ref[...],
                           preferred_element_type=jnp.float32)


TB = 2048
TBP = TB // 8


def _tc_body(r12_ref, m1_ref, w1blk_ref, dblk_ref, out_ref):
    r = r12_ref[...]
    out_ref[...] = (
        jnp.dot(r, w1blk_ref[...], preferred_element_type=jnp.float32)
        + jnp.dot(r * m1_ref[...], dblk_ref[...],
                  preferred_element_type=jnp.float32)
    )


def kernel(input, head_emb, head_W, tail_emb0, tail_W0, tail_emb1, tail_W1):
    flat = input.reshape(-1).astype(jnp.int32)
    t0part = jnp.transpose(
        tail_emb0.T.reshape(16, T0ROWS // 8, 8), (1, 2, 0)).reshape(T0ROWS // 8, 128)
    t1part = jnp.pad(
        jnp.transpose(tail_emb1.T.reshape(4, T1ROWS // 8, 8), (1, 2, 0)),
        ((0, 0), (0, 0), (0, 12))).reshape(T1ROWS // 8, 128)
    comb128 = jnp.concatenate([t0part, t1part], axis=0)

    m1f = ((flat >= C0) & (flat < C1)).astype(jnp.float32)
    m1p = jnp.repeat(m1f, 16).reshape(N // 8, 128)
    w1pad = jnp.pad(tail_W1, ((0, 12), (0, 0)))
    eye8 = jnp.eye(8, dtype=jnp.float32)
    w1blk = jnp.kron(eye8, w1pad)
    dblk = jnp.kron(eye8, tail_W0 - w1pad)

    p0 = pl.pallas_call(
        _tc_p0_body,
        grid=(10,),
        in_specs=[
            pl.BlockSpec((C0 // 10, EMB), lambda i: (i, 0)),
            pl.BlockSpec((EMB, EMB), lambda i: (0, 0)),
        ],
        out_specs=pl.BlockSpec((C0 // 10, EMB), lambda i: (i, 0)),
        out_shape=jax.ShapeDtypeStruct((C0, EMB), jnp.float32),
    )(head_emb, head_W)

    r12 = _sc_gather(flat, comb128.reshape(TROWS, 16))

    out2 = pl.pallas_call(
        _tc_body,
        grid=(N // TB,),
        in_specs=[
            pl.BlockSpec((TBP, 128), lambda i: (i, 0)),
            pl.BlockSpec((TBP, 128), lambda i: (i, 0)),
            pl.BlockSpec((128, 512), lambda i: (0, 0)),
            pl.BlockSpec((128, 512), lambda i: (0, 0)),
        ],
        out_specs=pl.BlockSpec((TBP, 512), lambda i: (i, 0)),
        out_shape=jax.ShapeDtypeStruct((N // 8, 512), jnp.float32),
    )(r12.reshape(N // 8, 128), m1p, w1blk, dblk)
    out = out2.reshape(N, EMB)

    ref = jax.new_ref(out)
    _sc_head_scatter(flat, p0, ref)
    return ref[...].reshape(input.shape[0], input.shape[1], EMB)


# speedup vs baseline: 44.9223x; 1.2404x over previous
import functools

import jax
import jax.numpy as jnp
from jax import lax
from jax.experimental import pallas as pl
from jax.experimental.pallas import tpu as pltpu
from jax.experimental.pallas import tpu_sc as plsc

EMB = 64
C0, C1, C2 = 20000, 200000, 1000000
T0ROWS = C1 - C0
T1ROWS = C2 - C1
TROWS = T0ROWS + T1ROWS
N = 4096 * 200

NC, NS = 2, 16
NW = NC * NS
TOK_W = N // NW
CH = 512
NCH = TOK_W // CH
HB = 128

_mesh = plsc.VectorSubcoreMesh(
    core_axis_name="c", subcore_axis_name="s", num_cores=NC, num_subcores=NS
)
_sc_params = pltpu.CompilerParams(use_tc_tiling_on_sc=False, needs_layout_passes=False)


@functools.partial(
    pl.kernel,
    mesh=_mesh,
    compiler_params=_sc_params,
    out_type=jax.ShapeDtypeStruct((N, 16), jnp.float32),
    scratch_types=[
        pltpu.VMEM((CH,), jnp.int32),
        pltpu.VMEM((CH,), jnp.int32),
        pltpu.VMEM((CH, 16), jnp.float32),
        pltpu.SemaphoreType.DMA,
    ],
)
def _sc_gather(flat_hbm, comb_hbm, r12_hbm, idx_v, lidx_v, rows_v, sem):
    wid = lax.axis_index("s") * NC + lax.axis_index("c")
    base_w = wid * TOK_W

    def chunk_body(g, carry):
        base = base_w + g * CH
        pltpu.sync_copy(flat_hbm.at[pl.ds(base, CH)], idx_v)

        def vec_body(j, c):
            v = idx_v[pl.ds(j * 16, 16)]
            lidx = jnp.where(v < C0, 0,
                             jnp.where(v < C1, v - C0, T0ROWS + (v - C1)))
            lidx_v[pl.ds(j * 16, 16)] = lidx
            return c

        lax.fori_loop(0, CH // 16, vec_body, 0)
        pltpu.async_copy(comb_hbm.at[lidx_v], rows_v, sem).wait()
        pltpu.sync_copy(rows_v, r12_hbm.at[pl.ds(base, CH)])
        return carry

    lax.fori_loop(0, NCH, chunk_body, 0)


LCAP = TOK_W + HB + 16
DUMP = LCAP - 1


@functools.partial(
    pl.kernel,
    mesh=_mesh,
    compiler_params=_sc_params,
    out_type=(),
    scratch_types=[
        pltpu.VMEM((CH,), jnp.int32),
        pltpu.VMEM((LCAP,), jnp.int32),
        pltpu.VMEM((LCAP,), jnp.int32),
        pltpu.VMEM((HB, EMB), jnp.float32),
        pltpu.SemaphoreType.DMA,
    ],
)
def _sc_head_scatter(flat_hbm, p0_hbm, out_ref, idx_v, pos_v, hidx_v, rows_v, sem):
    wid = lax.axis_index("s") * NC + lax.axis_index("c")
    base_w = wid * TOK_W

    def chunk_body(g, off):
        base = base_w + g * CH
        pltpu.sync_copy(flat_hbm.at[pl.ds(base, CH)], idx_v)

        def vec_body(j, off):
            v = idx_v[pl.ds(j * 16, 16)]
            m0 = v < C0
            c = jnp.where(m0, 1, 0)
            rank = plsc.cumsum(c) - c
            dst = jnp.where(m0, off + rank, DUMP)
            plsc.store_scatter(pos_v, [dst], base + j * 16 + lax.iota(jnp.int32, 16))
            plsc.store_scatter(hidx_v, [dst], v)
            return off + jnp.sum(c)

        return lax.fori_loop(0, CH // 16, vec_body, off)

    off = lax.fori_loop(0, NCH, chunk_body, 0)

    nb = (off + HB - 1) // HB
    offp = nb * HB

    @pl.when(off > 0)
    def _pad():
        lane0 = lax.iota(jnp.int32, 16) == 0
        pos0 = jnp.sum(jnp.where(lane0, pos_v[pl.ds(0, 16)], 0))
        hid0 = jnp.sum(jnp.where(lane0, hidx_v[pl.ds(0, 16)], 0))

        def pad_body(k, c):
            lanes = k * 16 + lax.iota(jnp.int32, 16)
            m = lanes >= off
            cur_p = pos_v[pl.ds(k * 16, 16)]
            cur_h = hidx_v[pl.ds(k * 16, 16)]
            pos_v[pl.ds(k * 16, 16)] = jnp.where(m, pos0, cur_p)
            hidx_v[pl.ds(k * 16, 16)] = jnp.where(m, hid0, cur_h)
            return c

        lax.fori_loop(off // 16, (offp + 15) // 16, pad_body, 0)

    def batch_body(b, c):
        s = pl.ds(b * HB, HB)
        pltpu.async_copy(p0_hbm.at[hidx_v.at[s]], rows_v, sem).wait()
        pltpu.async_copy(rows_v, out_ref.at[pos_v.at[s]], sem).wait()
        return c

    lax.fori_loop(0, nb, batch_body, 0)


def _tc_p0_body(emb_ref, w_ref, out_ref):
    out_ref[...] = jnp.dot(emb_ref[...], w_ref[...],
                           preferred_element_type=jnp.float32)


TB = 8192
TBP = TB // 8


def _tc_body(r12_ref, m1_ref, w1blk_ref, dblk_ref, out_ref):
    r = r12_ref[...]
    res = (
        jnp.dot(r, w1blk_ref[...], preferred_element_type=jnp.float32)
        + jnp.dot(r * m1_ref[...], dblk_ref[...],
                  preferred_element_type=jnp.float32)
    )
    out_ref[...] = res.reshape(TB // 2, 128)


def kernel(input, head_emb, head_W, tail_emb0, tail_W0, tail_emb1, tail_W1):
    flat = input.reshape(-1).astype(jnp.int32)
    t0part = jnp.transpose(
        tail_emb0.T.reshape(16, T0ROWS // 8, 8), (1, 2, 0)).reshape(T0ROWS // 8, 128)
    t1part = jnp.pad(
        jnp.transpose(tail_emb1.T.reshape(4, T1ROWS // 8, 8), (1, 2, 0)),
        ((0, 0), (0, 0), (0, 12))).reshape(T1ROWS // 8, 128)
    comb128 = jnp.concatenate([t0part, t1part], axis=0)

    m1f = ((flat >= C0) & (flat < C1)).astype(jnp.float32)
    m1p = jnp.repeat(m1f, 16).reshape(N // 8, 128)
    w1pad = jnp.pad(tail_W1, ((0, 12), (0, 0)))
    eye8 = jnp.eye(8, dtype=jnp.float32)
    w1blk = jnp.kron(eye8, w1pad)
    dblk = jnp.kron(eye8, tail_W0 - w1pad)

    p0 = pl.pallas_call(
        _tc_p0_body,
        grid=(10,),
        in_specs=[
            pl.BlockSpec((C0 // 10, EMB), lambda i: (i, 0)),
            pl.BlockSpec((EMB, EMB), lambda i: (0, 0)),
        ],
        out_specs=pl.BlockSpec((C0 // 10, EMB), lambda i: (i, 0)),
        out_shape=jax.ShapeDtypeStruct((C0, EMB), jnp.float32),
    )(head_emb, head_W)

    r12 = _sc_gather(flat, comb128.reshape(TROWS, 16))

    out2 = pl.pallas_call(
        _tc_body,
        grid=(N // TB,),
        in_specs=[
            pl.BlockSpec((TBP, 128), lambda i: (i, 0)),
            pl.BlockSpec((TBP, 128), lambda i: (i, 0)),
            pl.BlockSpec((128, 512), lambda i: (0, 0)),
            pl.BlockSpec((128, 512), lambda i: (0, 0)),
        ],
        out_specs=pl.BlockSpec((TB // 2, 128), lambda i: (i, 0)),
        out_shape=jax.ShapeDtypeStruct((N // 2, 128), jnp.float32),
    )(r12.reshape(N // 8, 128), m1p, w1blk, dblk)
    out = out2.reshape(N, EMB)

    ref = jax.new_ref(out)
    _sc_head_scatter(flat, p0, ref)
    return ref[...].reshape(input.shape[0], input.shape[1], EMB)


# speedup vs baseline: 46.2576x; 1.0297x over previous
import functools

import jax
import jax.numpy as jnp
from jax import lax
from jax.experimental import pallas as pl
from jax.experimental.pallas import tpu as pltpu
from jax.experimental.pallas import tpu_sc as plsc

EMB = 64
C0, C1, C2 = 20000, 200000, 1000000
T0ROWS = C1 - C0
T1ROWS = C2 - C1
TROWS = T0ROWS + T1ROWS
N = 4096 * 200

NC, NS = 2, 16
NW = NC * NS
TOK_W = N // NW
CH = 512
NCH = TOK_W // CH
HB = 128

_mesh = plsc.VectorSubcoreMesh(
    core_axis_name="c", subcore_axis_name="s", num_cores=NC, num_subcores=NS
)
_sc_params = pltpu.CompilerParams(use_tc_tiling_on_sc=False, needs_layout_passes=False)


@functools.partial(
    pl.kernel,
    mesh=_mesh,
    compiler_params=_sc_params,
    out_type=jax.ShapeDtypeStruct((N, 16), jnp.float32),
    scratch_types=[
        pltpu.VMEM((CH,), jnp.int32),
        pltpu.VMEM((CH,), jnp.int32),
        pltpu.VMEM((CH, 16), jnp.float32),
        pltpu.SemaphoreType.DMA,
    ],
)
def _sc_gather(flat_hbm, comb_hbm, r12_hbm, idx_v, lidx_v, rows_v, sem):
    wid = lax.axis_index("s") * NC + lax.axis_index("c")
    base_w = wid * TOK_W

    def chunk_body(g, carry):
        base = base_w + g * CH
        pltpu.sync_copy(flat_hbm.at[pl.ds(base, CH)], idx_v)

        def vec_body(j, c):
            v = idx_v[pl.ds(j * 16, 16)]
            lidx = jnp.where(v < C0, 0,
                             jnp.where(v < C1, v - C0, T0ROWS + (v - C1)))
            lidx_v[pl.ds(j * 16, 16)] = lidx
            return c

        lax.fori_loop(0, CH // 16, vec_body, 0)
        pltpu.async_copy(comb_hbm.at[lidx_v], rows_v, sem).wait()
        pltpu.sync_copy(rows_v, r12_hbm.at[pl.ds(base, CH)])
        return carry

    lax.fori_loop(0, NCH, chunk_body, 0)


LCAP = TOK_W + HB + 16
DUMP = LCAP - 1


@functools.partial(
    pl.kernel,
    mesh=_mesh,
    compiler_params=_sc_params,
    out_type=(),
    scratch_types=[
        pltpu.VMEM((CH,), jnp.int32),
        pltpu.VMEM((LCAP,), jnp.int32),
        pltpu.VMEM((LCAP,), jnp.int32),
        pltpu.VMEM((HB, EMB), jnp.float32),
        pltpu.SemaphoreType.DMA,
    ],
)
def _sc_head_scatter(flat_hbm, p0_hbm, out_ref, idx_v, pos_v, hidx_v, rows_v, sem):
    wid = lax.axis_index("s") * NC + lax.axis_index("c")
    base_w = wid * TOK_W

    def chunk_body(g, off):
        base = base_w + g * CH
        pltpu.sync_copy(flat_hbm.at[pl.ds(base, CH)], idx_v)

        def vec_body(j, off):
            v = idx_v[pl.ds(j * 16, 16)]
            m0 = v < C0
            c = jnp.where(m0, 1, 0)
            rank = plsc.cumsum(c) - c
            dst = jnp.where(m0, off + rank, DUMP)
            plsc.store_scatter(pos_v, [dst], base + j * 16 + lax.iota(jnp.int32, 16))
            plsc.store_scatter(hidx_v, [dst], v)
            return off + jnp.sum(c)

        return lax.fori_loop(0, CH // 16, vec_body, off)

    off = lax.fori_loop(0, NCH, chunk_body, 0)

    nb = (off + HB - 1) // HB
    offp = nb * HB

    @pl.when(off > 0)
    def _pad():
        lane0 = lax.iota(jnp.int32, 16) == 0
        pos0 = jnp.sum(jnp.where(lane0, pos_v[pl.ds(0, 16)], 0))
        hid0 = jnp.sum(jnp.where(lane0, hidx_v[pl.ds(0, 16)], 0))

        def pad_body(k, c):
            lanes = k * 16 + lax.iota(jnp.int32, 16)
            m = lanes >= off
            cur_p = pos_v[pl.ds(k * 16, 16)]
            cur_h = hidx_v[pl.ds(k * 16, 16)]
            pos_v[pl.ds(k * 16, 16)] = jnp.where(m, pos0, cur_p)
            hidx_v[pl.ds(k * 16, 16)] = jnp.where(m, hid0, cur_h)
            return c

        lax.fori_loop(off // 16, (offp + 15) // 16, pad_body, 0)

    def batch_body(b, c):
        s = pl.ds(b * HB, HB)
        pltpu.async_copy(p0_hbm.at[hidx_v.at[s]], rows_v, sem).wait()
        pltpu.async_copy(rows_v, out_ref.at[pos_v.at[s]], sem).wait()
        return c

    lax.fori_loop(0, nb, batch_body, 0)


def _tc_p0_body(emb_ref, w_ref, out_ref):
    out_ref[...] = jnp.dot(emb_ref[...], w_ref[...],
                           preferred_element_type=jnp.float32)


TB = 8192
TBP = TB // 8


def _tc_body(r12_ref, m1_ref, w1blk_ref, dblk_ref, out_ref):
    r = r12_ref[...]
    res = (
        jnp.dot(r, w1blk_ref[...], preferred_element_type=jnp.float32)
        + jnp.dot(r * m1_ref[...], dblk_ref[...],
                  preferred_element_type=jnp.float32)
    )
    out_ref[...] = res.reshape(TB // 2, 128)


def kernel(input, head_emb, head_W, tail_emb0, tail_W0, tail_emb1, tail_W1):
    flat = input.T.reshape(-1).astype(jnp.int32)
    t0part = jnp.transpose(
        tail_emb0.T.reshape(16, T0ROWS // 8, 8), (1, 2, 0)).reshape(T0ROWS // 8, 128)
    t1part = jnp.pad(
        jnp.transpose(tail_emb1.T.reshape(4, T1ROWS // 8, 8), (1, 2, 0)),
        ((0, 0), (0, 0), (0, 12))).reshape(T1ROWS // 8, 128)
    comb128 = jnp.concatenate([t0part, t1part], axis=0)

    m1f = ((flat >= C0) & (flat < C1)).astype(jnp.float32)
    m1p = jnp.repeat(m1f, 16).reshape(N // 8, 128)
    w1pad = jnp.pad(tail_W1, ((0, 12), (0, 0)))
    eye8 = jnp.eye(8, dtype=jnp.float32)
    w1blk = jnp.kron(eye8, w1pad)
    dblk = jnp.kron(eye8, tail_W0 - w1pad)

    p0 = pl.pallas_call(
        _tc_p0_body,
        grid=(10,),
        in_specs=[
            pl.BlockSpec((C0 // 10, EMB), lambda i: (i, 0)),
            pl.BlockSpec((EMB, EMB), lambda i: (0, 0)),
        ],
        out_specs=pl.BlockSpec((C0 // 10, EMB), lambda i: (i, 0)),
        out_shape=jax.ShapeDtypeStruct((C0, EMB), jnp.float32),
    )(head_emb, head_W)

    r12 = _sc_gather(flat, comb128.reshape(TROWS, 16))

    out2 = pl.pallas_call(
        _tc_body,
        grid=(N // TB,),
        in_specs=[
            pl.BlockSpec((TBP, 128), lambda i: (i, 0)),
            pl.BlockSpec((TBP, 128), lambda i: (i, 0)),
            pl.BlockSpec((128, 512), lambda i: (0, 0)),
            pl.BlockSpec((128, 512), lambda i: (0, 0)),
        ],
        out_specs=pl.BlockSpec((TB // 2, 128), lambda i: (i, 0)),
        out_shape=jax.ShapeDtypeStruct((N // 2, 128), jnp.float32),
    )(r12.reshape(N // 8, 128), m1p, w1blk, dblk)
    out = out2.reshape(N, EMB)

    ref = jax.new_ref(out)
    _sc_head_scatter(flat, p0, ref)
    return ref[...].reshape(input.shape[1], input.shape[0], EMB).swapaxes(0, 1)
